# Initial kernel scaffold; baseline (speedup 1.0000x reference)
import jax
import jax.numpy as jnp
from jax.experimental import pallas as pl


def kernel(x, weight, bias):
    raise NotImplementedError("write your pallas kernel here")


# speedup vs baseline: 1.5526x; 1.5526x over previous
import jax
import jax.numpy as jnp
from jax.experimental import pallas as pl
from jax.experimental.pallas import tpu as pltpu

_G = 32
_T = 5


def _dot(a, b):
    return jax.lax.dot_general(a, b, (((1,), (0,)), ((), ())),
                               preferred_element_type=jnp.float32)


def _whiten_kernel(x_ref, w_ref, b_ref, o_ref):
    G = _G
    m = x_ref.shape[2]
    K = w_ref.shape[1]
    L = m // K

    xr = x_ref[0]
    mu = jnp.sum(xr, axis=1, keepdims=True) * (1.0 / m)
    mu_t = jnp.transpose(mu)

    s_raw = jax.lax.dot_general(xr, xr, (((1,), (1,)), ((), ())),
                                preferred_element_type=jnp.float32)
    sigma = s_raw * (1.0 / m) - mu * mu_t

    rows = jax.lax.broadcasted_iota(jnp.int32, (G, G), 0)
    cols = jax.lax.broadcasted_iota(jnp.int32, (G, G), 1)
    eye = rows == cols

    trace = jnp.sum(jnp.where(eye, sigma, 0.0))
    trace_inv = 1.0 / trace
    sigma_n = sigma * trace_inv

    p = jnp.where(eye, 1.0, 0.0).astype(jnp.float32)
    for _ in range(_T):
        p3 = _dot(_dot(p, p), p)
        p = 1.5 * p - 0.5 * _dot(p3, sigma_n)

    wm = p * jax.lax.rsqrt(trace)
    wm_mu = jnp.sum(wm * mu_t, axis=1, keepdims=True)

    for k in range(K):
        wk = w_ref[:, k:k + 1]
        bk = b_ref[:, k:k + 1]
        wmk = wm * wk
        ck = bk - wk * wm_mu
        yk = _dot(wmk, xr[:, k * L:(k + 1) * L])
        o_ref[0, :, k * L:(k + 1) * L] = yk + ck


def kernel(x, weight, bias):
    B, C, L = x.shape
    G = _G
    K = C // G
    m = K * L

    xg = x.reshape(B, G, m)
    w2 = weight.reshape(C)[None, :].reshape(G, K)
    b2 = bias.reshape(C)[None, :].reshape(G, K)

    out = pl.pallas_call(
        _whiten_kernel,
        grid=(B,),
        in_specs=[
            pl.BlockSpec((1, G, m), lambda b: (b, 0, 0)),
            pl.BlockSpec((G, K), lambda b: (0, 0)),
            pl.BlockSpec((G, K), lambda b: (0, 0)),
        ],
        out_specs=pl.BlockSpec((1, G, m), lambda b: (b, 0, 0)),
        out_shape=jax.ShapeDtypeStruct((B, G, m), x.dtype),
        compiler_params=pltpu.CompilerParams(
            dimension_semantics=("parallel",),
        ),
    )(xg, w2, b2)
    return out.reshape(B, C, L)


# speedup vs baseline: 2.8359x; 1.8266x over previous
import jax
import jax.numpy as jnp
from jax.experimental import pallas as pl
from jax.experimental.pallas import tpu as pltpu

_G = 32
_T = 5


def _dot(a, b):
    return jax.lax.dot_general(a, b, (((1,), (0,)), ((), ())),
                               preferred_element_type=jnp.float32)


def _dot_t(a, b):
    return jax.lax.dot_general(a, b, (((1,), (1,)), ((), ())),
                               preferred_element_type=jnp.float32)


def _whiten_kernel(x_ref, w_ref, b_ref, o_ref):
    G = _G
    K = x_ref.shape[2]
    L = x_ref.shape[3]
    m = K * L

    rs = jnp.sum(x_ref[0], axis=2)
    mu = jnp.sum(rs, axis=1, keepdims=True) * (1.0 / m)
    mu_t = jnp.transpose(mu)

    s_raw = _dot_t(x_ref[0, :, 0, :], x_ref[0, :, 0, :])
    for k in range(1, K):
        xk = x_ref[0, :, k, :]
        s_raw = s_raw + _dot_t(xk, xk)
    sigma = s_raw * (1.0 / m) - mu * mu_t

    rows = jax.lax.broadcasted_iota(jnp.int32, (G, G), 0)
    cols = jax.lax.broadcasted_iota(jnp.int32, (G, G), 1)
    eye = rows == cols

    trace = jnp.sum(jnp.where(eye, sigma, 0.0))
    trace_inv = 1.0 / trace
    sigma_n = sigma * trace_inv

    p = jnp.where(eye, 1.0, 0.0).astype(jnp.float32)
    for _ in range(_T):
        p3 = _dot(_dot(p, p), p)
        p = 1.5 * p - 0.5 * _dot(p3, sigma_n)

    wm = p * jax.lax.rsqrt(trace)
    wm_mu = jnp.sum(wm * mu_t, axis=1, keepdims=True)

    for k in range(K):
        wk = w_ref[:, k:k + 1]
        bk = b_ref[:, k:k + 1]
        wmk = wm * wk
        ck = bk - wk * wm_mu
        yk = _dot(wmk, x_ref[0, :, k, :])
        o_ref[0, :, k, :] = yk + ck


def kernel(x, weight, bias):
    B, C, L = x.shape
    G = _G
    K = C // G

    x4 = x.reshape(B, G, K, L)
    w2 = weight.reshape(G, K)
    b2 = bias.reshape(G, K)

    out = pl.pallas_call(
        _whiten_kernel,
        grid=(B,),
        in_specs=[
            pl.BlockSpec((1, G, K, L), lambda b: (b, 0, 0, 0)),
            pl.BlockSpec((G, K), lambda b: (0, 0)),
            pl.BlockSpec((G, K), lambda b: (0, 0)),
        ],
        out_specs=pl.BlockSpec((1, G, K, L), lambda b: (b, 0, 0, 0)),
        out_shape=jax.ShapeDtypeStruct((B, G, K, L), x.dtype),
        compiler_params=pltpu.CompilerParams(
            dimension_semantics=("parallel",),
        ),
    )(x4, w2, b2)
    return out.reshape(B, C, L)


# speedup vs baseline: 4.1663x; 1.4691x over previous
import jax
import jax.numpy as jnp
from jax.experimental import pallas as pl
from jax.experimental.pallas import tpu as pltpu

_G = 32
_T = 5


def _dot(a, b):
    return jax.lax.dot_general(a, b, (((1,), (0,)), ((), ())),
                               preferred_element_type=jnp.float32)


def _dot_t(a, b):
    return jax.lax.dot_general(a, b, (((1,), (1,)), ((), ())),
                               preferred_element_type=jnp.float32)


def _dot_tl(a, b):
    return jax.lax.dot_general(a, b, (((0,), (0,)), ((), ())),
                               preferred_element_type=jnp.float32)


def _whiten_kernel(x_ref, w_ref, b_ref, o_ref):
    G = _G
    C = x_ref.shape[1]
    L = x_ref.shape[2]
    K = C // G
    m = K * L

    xr = x_ref[0]
    rs = jnp.sum(xr, axis=1, keepdims=True)
    cc = _dot_t(xr, xr)

    gidx = jax.lax.broadcasted_iota(jnp.int32, (C, G), 0) // K
    gcol = jax.lax.broadcasted_iota(jnp.int32, (C, G), 1)
    rmat = jnp.where(gidx == gcol, 1.0, 0.0)

    mrow = jax.lax.broadcasted_iota(jnp.int32, (C, C), 0) % K
    mcol = jax.lax.broadcasted_iota(jnp.int32, (C, C), 1) % K
    kdiag = jnp.where(mrow == mcol, 1.0, 0.0)
    ccm = cc * kdiag

    t1 = _dot(ccm, rmat)
    s_raw = _dot_tl(rmat, t1)
    gs = _dot_tl(rmat, rs)
    mu = gs * (1.0 / m)
    mu_t = jnp.transpose(mu)
    sigma = s_raw * (1.0 / m) - mu * mu_t

    rows = jax.lax.broadcasted_iota(jnp.int32, (G, G), 0)
    cols = jax.lax.broadcasted_iota(jnp.int32, (G, G), 1)
    eye = rows == cols

    trace = jnp.sum(jnp.where(eye, sigma, 0.0))
    trace_inv = 1.0 / trace
    sigma_n = sigma * trace_inv

    p = jnp.where(eye, 1.0, 0.0).astype(jnp.float32)
    for _ in range(_T):
        p3 = _dot(_dot(p, p), p)
        p = 1.5 * p - 0.5 * _dot(p3, sigma_n)

    wm = p * jax.lax.rsqrt(trace)

    t2 = _dot(rmat, wm) * w_ref[...]
    wbig = _dot_t(t2, rmat) * kdiag

    wm_mu = jnp.sum(wm * mu_t, axis=1, keepdims=True)
    cvec = b_ref[...] - w_ref[...] * _dot(rmat, wm_mu)

    o_ref[0] = _dot(wbig, xr) + cvec


def kernel(x, weight, bias):
    B, C, L = x.shape

    w2 = weight.reshape(C, 1)
    b2 = bias.reshape(C, 1)

    return pl.pallas_call(
        _whiten_kernel,
        grid=(B,),
        in_specs=[
            pl.BlockSpec((1, C, L), lambda b: (b, 0, 0)),
            pl.BlockSpec((C, 1), lambda b: (0, 0)),
            pl.BlockSpec((C, 1), lambda b: (0, 0)),
        ],
        out_specs=pl.BlockSpec((1, C, L), lambda b: (b, 0, 0)),
        out_shape=jax.ShapeDtypeStruct((B, C, L), x.dtype),
        compiler_params=pltpu.CompilerParams(
            dimension_semantics=("parallel",),
        ),
    )(x, w2, b2)


# speedup vs baseline: 4.4565x; 1.0697x over previous
import jax
import jax.numpy as jnp
from jax.experimental import pallas as pl
from jax.experimental.pallas import tpu as pltpu

_G = 32
_T = 5


def _dot(a, b):
    return jax.lax.dot_general(a, b, (((1,), (0,)), ((), ())),
                               preferred_element_type=jnp.float32)


def _dot_t(a, b):
    return jax.lax.dot_general(a, b, (((1,), (1,)), ((), ())),
                               preferred_element_type=jnp.float32)


def _dot_tl(a, b):
    return jax.lax.dot_general(a, b, (((0,), (0,)), ((), ())),
                               preferred_element_type=jnp.float32)


def _whiten_kernel(x_ref, w_ref, b_ref, o_ref):
    G = _G
    C = x_ref.shape[1]
    L = x_ref.shape[2]
    K = C // G
    m = K * L

    xr = x_ref[0]
    rs = jnp.sum(xr, axis=1, keepdims=True)
    cc = _dot_t(xr, xr)

    gidx = jax.lax.broadcasted_iota(jnp.int32, (C, G), 0) // K
    gcol = jax.lax.broadcasted_iota(jnp.int32, (C, G), 1)
    rmat = jnp.where(gidx == gcol, 1.0, 0.0)

    mrow = jax.lax.broadcasted_iota(jnp.int32, (C, C), 0) % K
    mcol = jax.lax.broadcasted_iota(jnp.int32, (C, C), 1) % K
    kdiag = jnp.where(mrow == mcol, 1.0, 0.0)
    ccm = cc * kdiag

    t1 = _dot(ccm, rmat)
    s_raw = _dot_tl(rmat, t1)
    gs = _dot_tl(rmat, rs)
    mu = gs * (1.0 / m)
    mu_t = jnp.transpose(mu)
    sigma = s_raw * (1.0 / m) - mu * mu_t

    rows = jax.lax.broadcasted_iota(jnp.int32, (G, G), 0)
    cols = jax.lax.broadcasted_iota(jnp.int32, (G, G), 1)
    eye = rows == cols

    trace = jnp.sum(jnp.where(eye, sigma, 0.0))
    trace_inv = 1.0 / trace
    sigma_n = sigma * trace_inv

    eyef = jnp.where(eye, 1.0, 0.0).astype(jnp.float32)
    p = 1.5 * eyef - 0.5 * sigma_n
    for _ in range(_T - 1):
        pp = _dot(p, p)
        ps = _dot(p, sigma_n)
        p = 1.5 * p - 0.5 * _dot(pp, ps)

    wm = p * jax.lax.rsqrt(trace)

    t2 = _dot(rmat, wm) * w_ref[...]
    wbig = _dot_t(t2, rmat) * kdiag

    wm_mu = jnp.sum(wm * mu_t, axis=1, keepdims=True)
    cvec = b_ref[...] - w_ref[...] * _dot(rmat, wm_mu)

    o_ref[0] = _dot(wbig, xr) + cvec


def kernel(x, weight, bias):
    B, C, L = x.shape

    w2 = weight.reshape(C, 1)
    b2 = bias.reshape(C, 1)

    return pl.pallas_call(
        _whiten_kernel,
        grid=(B,),
        in_specs=[
            pl.BlockSpec((1, C, L), lambda b: (b, 0, 0)),
            pl.BlockSpec((C, 1), lambda b: (0, 0)),
            pl.BlockSpec((C, 1), lambda b: (0, 0)),
        ],
        out_specs=pl.BlockSpec((1, C, L), lambda b: (b, 0, 0)),
        out_shape=jax.ShapeDtypeStruct((B, C, L), x.dtype),
        compiler_params=pltpu.CompilerParams(
            dimension_semantics=("parallel",),
        ),
    )(x, w2, b2)
